# baseline (speedup 1.0000x reference)
import jax
import jax.numpy as jnp
from jax.experimental import pallas as pl
from jax.experimental.pallas import tpu as pltpu

_NUM_EXPERTS = 64
_BLOCK_TOKENS = 4096


def _router_kernel(x_ref, w_ref, probs_ref, w_out_ref, i_out_ref):
    x = x_ref[0]
    logits = jax.lax.dot_general(
        w_ref[...], x,
        dimension_numbers=(((1,), (1,)), ((), ())),
        preferred_element_type=jnp.float32,
    )
    m1 = jnp.max(logits, axis=0, keepdims=True)
    e = jnp.exp(logits - m1)
    s = jnp.sum(e, axis=0, keepdims=True)
    probs_ref[0] = e / s

    iota = jax.lax.broadcasted_iota(jnp.int32, logits.shape, 0)
    i1 = jnp.min(jnp.where(logits == m1, iota, _NUM_EXPERTS), axis=0)
    masked = jnp.where(iota == i1[None, :], -jnp.inf, logits)
    m2 = jnp.max(masked, axis=0)
    i2 = jnp.min(jnp.where(masked == m2[None, :], iota, _NUM_EXPERTS), axis=0)

    r = jnp.exp(m2 - m1[0])
    w1 = 1.0 / (1.0 + r)
    w2 = r / (1.0 + r)
    w_out_ref[0] = jnp.stack([w1, w2], axis=0)
    i_out_ref[0] = jnp.stack([i1, i2], axis=0).astype(jnp.int32)


@jax.jit
def kernel(x, W):
    b, s, d = x.shape
    grid = (b, s // _BLOCK_TOKENS)
    probs_t, weights_t, indices_t = pl.pallas_call(
        _router_kernel,
        grid=grid,
        in_specs=[
            pl.BlockSpec((1, _BLOCK_TOKENS, d), lambda i, j: (i, j, 0)),
            pl.BlockSpec((_NUM_EXPERTS, d), lambda i, j: (0, 0)),
        ],
        out_specs=[
            pl.BlockSpec((1, _NUM_EXPERTS, _BLOCK_TOKENS), lambda i, j: (i, 0, j)),
            pl.BlockSpec((1, 2, _BLOCK_TOKENS), lambda i, j: (i, 0, j)),
            pl.BlockSpec((1, 2, _BLOCK_TOKENS), lambda i, j: (i, 0, j)),
        ],
        out_shape=[
            jax.ShapeDtypeStruct((b, _NUM_EXPERTS, s), jnp.float32),
            jax.ShapeDtypeStruct((b, 2, s), jnp.float32),
            jax.ShapeDtypeStruct((b, 2, s), jnp.int32),
        ],
        compiler_params=pltpu.CompilerParams(
            dimension_semantics=("parallel", "parallel"),
        ),
    )(x, W)
    return (
        jnp.transpose(weights_t, (0, 2, 1)),
        jnp.transpose(indices_t, (0, 2, 1)),
        jnp.transpose(probs_t, (0, 2, 1)),
    )
